# baseline (device time: 90542 ns/iter reference)
import jax
import jax.numpy as jnp
from jax import lax
from jax.experimental import pallas as pl
from jax.experimental.pallas import tpu as pltpu

N_DEV = 4
N_EXPERTS = 16


def kernel(x, router_W, route_idx, expert_W):
    m, d = x.shape
    n_loc, _, h = expert_W.shape
    m2 = m // 2
    n_hops = N_DEV - 1

    def body(x_ref, rw_ref, ri_ref, ew_ref, out_ref,
             xin_bf, g_own, ew_bf,
             xc_cw, xc_ccw, gc_cw, gc_ccw,
             rsb_cw, rsb_ccw, rss_cw, rss_ccw,
             agx_cw_s, agx_cw_r, agx_ccw_s, agx_ccw_r,
             agg_cw_s, agg_cw_r, agg_ccw_s, agg_ccw_r,
             rs_cw_s, rs_cw_r, rs_ccw_s, rs_ccw_r):
        my = lax.axis_index("i")
        right = lax.rem(my + 1, N_DEV)
        left = lax.rem(my + N_DEV - 1, N_DEV)

        barrier = pltpu.get_barrier_semaphore()
        pl.semaphore_signal(barrier, inc=1, device_id=(left,),
                            device_id_type=pl.DeviceIdType.MESH)
        pl.semaphore_signal(barrier, inc=1, device_id=(right,),
                            device_id_type=pl.DeviceIdType.MESH)
        pl.semaphore_wait(barrier, 2)

        def contrib(x_bf, g_blk):
            rows = g_blk.shape[0]
            le_iota = lax.broadcasted_iota(jnp.int32, (rows, N_EXPERTS), 1)
            parts = []
            for le in range(n_loc):
                eg = my * n_loc + le
                w = jnp.sum(jnp.where(le_iota == eg, g_blk, 0.0),
                            axis=-1, keepdims=True)
                parts.append(w.astype(jnp.bfloat16) * x_bf)
            x4 = jnp.concatenate(parts, axis=1)
            return jnp.dot(x4, ew_bf[:, :],
                           preferred_element_type=jnp.float32)

        agx_cw = [
            pltpu.make_async_remote_copy(
                src_ref=(xin_bf.at[0:m2] if hh == 0 else xc_cw.at[hh - 1]),
                dst_ref=xc_cw.at[hh],
                send_sem=agx_cw_s.at[hh], recv_sem=agx_cw_r.at[hh],
                device_id=(right,), device_id_type=pl.DeviceIdType.MESH)
            for hh in range(n_hops)
        ]
        agg_cw = [
            pltpu.make_async_remote_copy(
                src_ref=(g_own.at[0:m2] if hh == 0 else gc_cw.at[hh - 1]),
                dst_ref=gc_cw.at[hh],
                send_sem=agg_cw_s.at[hh], recv_sem=agg_cw_r.at[hh],
                device_id=(right,), device_id_type=pl.DeviceIdType.MESH)
            for hh in range(n_hops)
        ]
        agx_ccw = [
            pltpu.make_async_remote_copy(
                src_ref=(xin_bf.at[m2:m] if hh == 0 else xc_ccw.at[hh - 1]),
                dst_ref=xc_ccw.at[hh],
                send_sem=agx_ccw_s.at[hh], recv_sem=agx_ccw_r.at[hh],
                device_id=(left,), device_id_type=pl.DeviceIdType.MESH)
            for hh in range(n_hops)
        ]
        agg_ccw = [
            pltpu.make_async_remote_copy(
                src_ref=(g_own.at[m2:m] if hh == 0 else gc_ccw.at[hh - 1]),
                dst_ref=gc_ccw.at[hh],
                send_sem=agg_ccw_s.at[hh], recv_sem=agg_ccw_r.at[hh],
                device_id=(left,), device_id_type=pl.DeviceIdType.MESH)
            for hh in range(n_hops)
        ]
        rs_cw = [
            pltpu.make_async_remote_copy(
                src_ref=rss_cw.at[hh], dst_ref=rsb_cw.at[hh],
                send_sem=rs_cw_s.at[hh], recv_sem=rs_cw_r.at[hh],
                device_id=(right,), device_id_type=pl.DeviceIdType.MESH)
            for hh in range(n_hops)
        ]
        rs_ccw = [
            pltpu.make_async_remote_copy(
                src_ref=rss_ccw.at[hh], dst_ref=rsb_ccw.at[hh],
                send_sem=rs_ccw_s.at[hh], recv_sem=rs_ccw_r.at[hh],
                device_id=(left,), device_id_type=pl.DeviceIdType.MESH)
            for hh in range(n_hops)
        ]

        xin_bf[:, :] = x_ref[:, :].astype(jnp.bfloat16)
        agx_cw[0].start()
        agx_ccw[0].start()

        e_iota = lax.broadcasted_iota(jnp.int32, (m, N_EXPERTS), 1)
        scores = jnp.dot(x_ref[:, :], rw_ref[:, :],
                         preferred_element_type=jnp.float32)
        p = jnp.exp(scores - jnp.max(scores, axis=-1, keepdims=True))
        sel = (e_iota == ri_ref[:, 0:1]) | (e_iota == ri_ref[:, 1:2])
        pm = jnp.where(sel, p, 0.0)
        g_own[:, :] = pm / jnp.sum(pm, axis=-1, keepdims=True)
        agg_cw[0].start()
        agg_ccw[0].start()

        for le in range(n_loc):
            ew_bf[le * d:(le + 1) * d, :] = ew_ref[le].astype(jnp.bfloat16)

        for s in range(n_hops):
            agx_cw[s].wait_recv()
            agg_cw[s].wait_recv()
            if s + 1 < n_hops:
                agx_cw[s + 1].start()
                agg_cw[s + 1].start()
            t = contrib(xc_cw[s], gc_cw[s])
            if s == 0:
                rss_cw[s, :, :] = t.astype(jnp.bfloat16)
            else:
                rs_cw[s - 1].wait_recv()
                rss_cw[s, :, :] = (
                    t + rsb_cw[s - 1].astype(jnp.float32)
                ).astype(jnp.bfloat16)
            rs_cw[s].start()

            agx_ccw[s].wait_recv()
            agg_ccw[s].wait_recv()
            if s + 1 < n_hops:
                agx_ccw[s + 1].start()
                agg_ccw[s + 1].start()
            t = contrib(xc_ccw[s], gc_ccw[s])
            if s == 0:
                rss_ccw[s, :, :] = t.astype(jnp.bfloat16)
            else:
                rs_ccw[s - 1].wait_recv()
                rss_ccw[s, :, :] = (
                    t + rsb_ccw[s - 1].astype(jnp.float32)
                ).astype(jnp.bfloat16)
            rs_ccw[s].start()

        own_lo = contrib(xin_bf[0:m2, :], g_own[0:m2, :])
        rs_cw[n_hops - 1].wait_recv()
        out_ref[0:m2, :] = own_lo + rsb_cw[n_hops - 1].astype(jnp.float32)

        own_hi = contrib(xin_bf[m2:m, :], g_own[m2:m, :])
        rs_ccw[n_hops - 1].wait_recv()
        out_ref[m2:m, :] = own_hi + rsb_ccw[n_hops - 1].astype(jnp.float32)

        for s in range(n_hops):
            agx_cw[s].wait_send()
            agg_cw[s].wait_send()
            agx_ccw[s].wait_send()
            agg_ccw[s].wait_send()
            rs_cw[s].wait_send()
            rs_ccw[s].wait_send()

    dma = pltpu.SemaphoreType.DMA((n_hops,))
    return pl.pallas_call(
        body,
        out_shape=jax.ShapeDtypeStruct((m, h), jnp.float32),
        in_specs=[pl.BlockSpec(memory_space=pltpu.VMEM)] * 4,
        out_specs=pl.BlockSpec(memory_space=pltpu.VMEM),
        scratch_shapes=[
            pltpu.VMEM((m, d), jnp.bfloat16),
            pltpu.VMEM((m, N_EXPERTS), jnp.float32),
            pltpu.VMEM((n_loc * d, h), jnp.bfloat16),
            pltpu.VMEM((n_hops, m2, d), jnp.bfloat16),
            pltpu.VMEM((n_hops, m2, d), jnp.bfloat16),
            pltpu.VMEM((n_hops, m2, N_EXPERTS), jnp.float32),
            pltpu.VMEM((n_hops, m2, N_EXPERTS), jnp.float32),
            pltpu.VMEM((n_hops, m2, h), jnp.bfloat16),
            pltpu.VMEM((n_hops, m2, h), jnp.bfloat16),
            pltpu.VMEM((n_hops, m2, h), jnp.bfloat16),
            pltpu.VMEM((n_hops, m2, h), jnp.bfloat16),
            dma, dma, dma, dma,
            dma, dma, dma, dma,
            dma, dma, dma, dma,
        ],
        compiler_params=pltpu.CompilerParams(
            collective_id=0,
            vmem_limit_bytes=60 * 1024 * 1024,
        ),
    )(x, router_W, route_idx, expert_W)


# device time: 84892 ns/iter; 1.0666x vs baseline; 1.0666x over previous
import jax
import jax.numpy as jnp
from jax import lax
from jax.experimental import pallas as pl
from jax.experimental.pallas import tpu as pltpu

N_DEV = 4
N_EXPERTS = 16


def kernel(x, router_W, route_idx, expert_W):
    m, d = x.shape
    n_loc, _, h = expert_W.shape
    m2 = m // 2
    n_hops = N_DEV - 1

    def body(x_ref, rw_ref, ri_ref, ew_ref, out_ref,
             xin_bf, g_own, ew_bf,
             xc_cw, xc_ccw, gc_cw, gc_ccw,
             rsb_cw, rsb_ccw, rss_cw, rss_ccw,
             agx_cw_s, agx_cw_r, agx_ccw_s, agx_ccw_r,
             agg_cw_s, agg_cw_r, agg_ccw_s, agg_ccw_r,
             rs_cw_s, rs_cw_r, rs_ccw_s, rs_ccw_r):
        my = lax.axis_index("i")
        right = lax.rem(my + 1, N_DEV)
        left = lax.rem(my + N_DEV - 1, N_DEV)

        barrier = pltpu.get_barrier_semaphore()
        pl.semaphore_signal(barrier, inc=1, device_id=(left,),
                            device_id_type=pl.DeviceIdType.MESH)
        pl.semaphore_signal(barrier, inc=1, device_id=(right,),
                            device_id_type=pl.DeviceIdType.MESH)
        pl.semaphore_wait(barrier, 2)

        def contrib(x_bf, g_blk):
            rows = g_blk.shape[0]
            le_iota = lax.broadcasted_iota(jnp.int32, (rows, N_EXPERTS), 1)
            acc = None
            for le in range(n_loc):
                eg = my * n_loc + le
                w = jnp.sum(jnp.where(le_iota == eg, g_blk, 0.0),
                            axis=-1, keepdims=True)
                mm = jnp.dot(x_bf, ew_bf[le * d:(le + 1) * d, :],
                             preferred_element_type=jnp.float32)
                t = w * mm
                acc = t if acc is None else acc + t
            return acc

        agx_cw = [
            pltpu.make_async_remote_copy(
                src_ref=(xin_bf.at[0:m2] if hh == 0 else xc_cw.at[hh - 1]),
                dst_ref=xc_cw.at[hh],
                send_sem=agx_cw_s.at[hh], recv_sem=agx_cw_r.at[hh],
                device_id=(right,), device_id_type=pl.DeviceIdType.MESH)
            for hh in range(n_hops)
        ]
        agg_cw = [
            pltpu.make_async_remote_copy(
                src_ref=(g_own.at[0:m2] if hh == 0 else gc_cw.at[hh - 1]),
                dst_ref=gc_cw.at[hh],
                send_sem=agg_cw_s.at[hh], recv_sem=agg_cw_r.at[hh],
                device_id=(right,), device_id_type=pl.DeviceIdType.MESH)
            for hh in range(n_hops)
        ]
        agx_ccw = [
            pltpu.make_async_remote_copy(
                src_ref=(xin_bf.at[m2:m] if hh == 0 else xc_ccw.at[hh - 1]),
                dst_ref=xc_ccw.at[hh],
                send_sem=agx_ccw_s.at[hh], recv_sem=agx_ccw_r.at[hh],
                device_id=(left,), device_id_type=pl.DeviceIdType.MESH)
            for hh in range(n_hops)
        ]
        agg_ccw = [
            pltpu.make_async_remote_copy(
                src_ref=(g_own.at[m2:m] if hh == 0 else gc_ccw.at[hh - 1]),
                dst_ref=gc_ccw.at[hh],
                send_sem=agg_ccw_s.at[hh], recv_sem=agg_ccw_r.at[hh],
                device_id=(left,), device_id_type=pl.DeviceIdType.MESH)
            for hh in range(n_hops)
        ]
        h2 = h // 2
        rs_cw = [
            [pltpu.make_async_remote_copy(
                src_ref=rss_cw.at[hh, slice(None), pl.ds(k * h2, h2)],
                dst_ref=rsb_cw.at[hh, slice(None), pl.ds(k * h2, h2)],
                send_sem=rs_cw_s.at[hh, k], recv_sem=rs_cw_r.at[hh, k],
                device_id=(right,), device_id_type=pl.DeviceIdType.MESH)
             for k in range(2)]
            for hh in range(n_hops)
        ]
        rs_ccw = [
            [pltpu.make_async_remote_copy(
                src_ref=rss_ccw.at[hh, slice(None), pl.ds(k * h2, h2)],
                dst_ref=rsb_ccw.at[hh, slice(None), pl.ds(k * h2, h2)],
                send_sem=rs_ccw_s.at[hh, k], recv_sem=rs_ccw_r.at[hh, k],
                device_id=(left,), device_id_type=pl.DeviceIdType.MESH)
             for k in range(2)]
            for hh in range(n_hops)
        ]

        xin_bf[:, :] = x_ref[:, :].astype(jnp.bfloat16)
        agx_cw[0].start()
        agx_ccw[0].start()

        e_iota = lax.broadcasted_iota(jnp.int32, (m, N_EXPERTS), 1)
        scores = jnp.dot(x_ref[:, :], rw_ref[:, :],
                         preferred_element_type=jnp.float32)
        p = jnp.exp(scores - jnp.max(scores, axis=-1, keepdims=True))
        sel = (e_iota == ri_ref[:, 0:1]) | (e_iota == ri_ref[:, 1:2])
        pm = jnp.where(sel, p, 0.0)
        g_own[:, :] = pm / jnp.sum(pm, axis=-1, keepdims=True)
        agg_cw[0].start()
        agg_ccw[0].start()

        for le in range(n_loc):
            ew_bf[le * d:(le + 1) * d, :] = ew_ref[le].astype(jnp.bfloat16)

        for s in range(n_hops):
            agx_cw[s].wait_recv()
            agg_cw[s].wait_recv()
            if s + 1 < n_hops:
                agx_cw[s + 1].start()
                agg_cw[s + 1].start()
            t = contrib(xc_cw[s], gc_cw[s])
            for k in range(2):
                ks = slice(k * h2, (k + 1) * h2)
                if s == 0:
                    rss_cw[s, :, ks] = t[:, ks].astype(jnp.bfloat16)
                else:
                    rs_cw[s - 1][k].wait_recv()
                    rss_cw[s, :, ks] = (
                        t[:, ks] + rsb_cw[s - 1, :, ks].astype(jnp.float32)
                    ).astype(jnp.bfloat16)
                rs_cw[s][k].start()

            agx_ccw[s].wait_recv()
            agg_ccw[s].wait_recv()
            if s + 1 < n_hops:
                agx_ccw[s + 1].start()
                agg_ccw[s + 1].start()
            t = contrib(xc_ccw[s], gc_ccw[s])
            for k in range(2):
                ks = slice(k * h2, (k + 1) * h2)
                if s == 0:
                    rss_ccw[s, :, ks] = t[:, ks].astype(jnp.bfloat16)
                else:
                    rs_ccw[s - 1][k].wait_recv()
                    rss_ccw[s, :, ks] = (
                        t[:, ks] + rsb_ccw[s - 1, :, ks].astype(jnp.float32)
                    ).astype(jnp.bfloat16)
                rs_ccw[s][k].start()

        own_lo = contrib(xin_bf[0:m2, :], g_own[0:m2, :])
        rs_cw[n_hops - 1][0].wait_recv()
        rs_cw[n_hops - 1][1].wait_recv()
        out_ref[0:m2, :] = own_lo + rsb_cw[n_hops - 1].astype(jnp.float32)

        own_hi = contrib(xin_bf[m2:m, :], g_own[m2:m, :])
        rs_ccw[n_hops - 1][0].wait_recv()
        rs_ccw[n_hops - 1][1].wait_recv()
        out_ref[m2:m, :] = own_hi + rsb_ccw[n_hops - 1].astype(jnp.float32)

        for s in range(n_hops):
            agx_cw[s].wait_send()
            agg_cw[s].wait_send()
            agx_ccw[s].wait_send()
            agg_ccw[s].wait_send()
            for k in range(2):
                rs_cw[s][k].wait_send()
                rs_ccw[s][k].wait_send()

    dma = pltpu.SemaphoreType.DMA((n_hops,))
    dma2 = pltpu.SemaphoreType.DMA((n_hops, 2))
    return pl.pallas_call(
        body,
        out_shape=jax.ShapeDtypeStruct((m, h), jnp.float32),
        in_specs=[pl.BlockSpec(memory_space=pltpu.VMEM)] * 4,
        out_specs=pl.BlockSpec(memory_space=pltpu.VMEM),
        scratch_shapes=[
            pltpu.VMEM((m, d), jnp.bfloat16),
            pltpu.VMEM((m, N_EXPERTS), jnp.float32),
            pltpu.VMEM((n_loc * d, h), jnp.bfloat16),
            pltpu.VMEM((n_hops, m2, d), jnp.bfloat16),
            pltpu.VMEM((n_hops, m2, d), jnp.bfloat16),
            pltpu.VMEM((n_hops, m2, N_EXPERTS), jnp.float32),
            pltpu.VMEM((n_hops, m2, N_EXPERTS), jnp.float32),
            pltpu.VMEM((n_hops, m2, h), jnp.bfloat16),
            pltpu.VMEM((n_hops, m2, h), jnp.bfloat16),
            pltpu.VMEM((n_hops, m2, h), jnp.bfloat16),
            pltpu.VMEM((n_hops, m2, h), jnp.bfloat16),
            dma, dma, dma, dma,
            dma, dma, dma, dma,
            dma2, dma2, dma2, dma2,
        ],
        compiler_params=pltpu.CompilerParams(
            collective_id=0,
            vmem_limit_bytes=60 * 1024 * 1024,
        ),
    )(x, router_W, route_idx, expert_W)
